# baseline (speedup 1.0000x reference)
import functools

import jax
import jax.numpy as jnp
from jax.experimental import pallas as pl

HIDDEN = 2048
NUM_EXPERTS = 8
TOP_K = 2
BLOCK_ROWS = 512


def _router_block(x_ref, w_ref, probs_ref, idx_ref, wts_ref):
    logits_t = jax.lax.dot_general(
        w_ref[...], x_ref[...],
        dimension_numbers=(((1,), (1,)), ((), ())),
        preferred_element_type=jnp.float32,
    )
    m = jnp.max(logits_t, axis=0, keepdims=True)
    e = jnp.exp(logits_t - m)
    s = jnp.sum(e, axis=0, keepdims=True)
    probs_t = e / s

    iota = jax.lax.broadcasted_iota(jnp.int32, probs_t.shape, 0)
    v1 = jnp.max(probs_t, axis=0, keepdims=True)
    i1 = jnp.min(jnp.where(probs_t == v1, iota, NUM_EXPERTS), axis=0,
                 keepdims=True)
    masked = jnp.where(iota == i1, -jnp.inf, probs_t)
    v2 = jnp.max(masked, axis=0, keepdims=True)
    i2 = jnp.min(jnp.where(masked == v2, iota, NUM_EXPERTS), axis=0,
                 keepdims=True)

    probs_ref[...] = probs_t
    idx_ref[...] = jnp.concatenate([i1, i2], axis=0)
    denom = v1 + v2
    wts_ref[...] = jnp.concatenate([v1 / denom, v2 / denom], axis=0)


@functools.partial(jax.jit, static_argnames=("interpret",))
def kernel(hidden_states, gate_w, interpret=False):
    b, s, h = hidden_states.shape
    n = b * s
    x = hidden_states.reshape(n, h)

    grid = (n // BLOCK_ROWS,)
    probs_t, idx_t, wts_t = pl.pallas_call(
        _router_block,
        grid=grid,
        in_specs=[
            pl.BlockSpec((BLOCK_ROWS, h), lambda i: (i, 0)),
            pl.BlockSpec((NUM_EXPERTS, h), lambda i: (0, 0)),
        ],
        out_specs=[
            pl.BlockSpec((NUM_EXPERTS, BLOCK_ROWS), lambda i: (0, i)),
            pl.BlockSpec((TOP_K, BLOCK_ROWS), lambda i: (0, i)),
            pl.BlockSpec((TOP_K, BLOCK_ROWS), lambda i: (0, i)),
        ],
        out_shape=[
            jax.ShapeDtypeStruct((NUM_EXPERTS, n), jnp.float32),
            jax.ShapeDtypeStruct((TOP_K, n), jnp.int32),
            jax.ShapeDtypeStruct((TOP_K, n), jnp.float32),
        ],
        interpret=interpret,
    )(x, gate_w)

    return (
        probs_t.T.reshape(b, s, NUM_EXPERTS),
        idx_t.T.reshape(b, s, TOP_K),
        wts_t.T.reshape(b, s, TOP_K),
    )


# speedup vs baseline: 1.2205x; 1.2205x over previous
import functools

import jax
import jax.numpy as jnp
from jax.experimental import pallas as pl
from jax.experimental.pallas import tpu as pltpu

HIDDEN = 2048
NUM_EXPERTS = 8
TOP_K = 2
BLOCK_ROWS = 1024


def _router_block(x_ref, w_ref, probs_ref, idx_ref, wts_ref):
    logits_t = jax.lax.dot_general(
        w_ref[...], x_ref[...],
        dimension_numbers=(((1,), (1,)), ((), ())),
        preferred_element_type=jnp.float32,
    )
    m = jnp.max(logits_t, axis=0, keepdims=True)
    e = jnp.exp(logits_t - m)
    s = jnp.sum(e, axis=0, keepdims=True)
    probs_t = e / s

    iota = jax.lax.broadcasted_iota(jnp.int32, probs_t.shape, 0)
    v1 = jnp.max(probs_t, axis=0, keepdims=True)
    i1 = jnp.min(jnp.where(probs_t == v1, iota, NUM_EXPERTS), axis=0,
                 keepdims=True)
    masked = jnp.where(iota == i1, -jnp.inf, probs_t)
    v2 = jnp.max(masked, axis=0, keepdims=True)
    i2 = jnp.min(jnp.where(masked == v2, iota, NUM_EXPERTS), axis=0,
                 keepdims=True)

    probs_ref[...] = probs_t
    idx_ref[...] = jnp.concatenate([i1, i2], axis=0)
    denom = v1 + v2
    wts_ref[...] = jnp.concatenate([v1 / denom, v2 / denom], axis=0)


@functools.partial(jax.jit, static_argnames=("interpret",))
def kernel(hidden_states, gate_w, interpret=False):
    b, s, h = hidden_states.shape
    n = b * s
    x = hidden_states.reshape(n, h)

    grid = (n // BLOCK_ROWS,)
    probs_t, idx_t, wts_t = pl.pallas_call(
        _router_block,
        grid=grid,
        in_specs=[
            pl.BlockSpec((BLOCK_ROWS, h), lambda i: (i, 0)),
            pl.BlockSpec((NUM_EXPERTS, h), lambda i: (0, 0)),
        ],
        out_specs=[
            pl.BlockSpec((NUM_EXPERTS, BLOCK_ROWS), lambda i: (0, i)),
            pl.BlockSpec((TOP_K, BLOCK_ROWS), lambda i: (0, i)),
            pl.BlockSpec((TOP_K, BLOCK_ROWS), lambda i: (0, i)),
        ],
        out_shape=[
            jax.ShapeDtypeStruct((NUM_EXPERTS, n), jnp.float32),
            jax.ShapeDtypeStruct((TOP_K, n), jnp.int32),
            jax.ShapeDtypeStruct((TOP_K, n), jnp.float32),
        ],
        compiler_params=pltpu.CompilerParams(
            dimension_semantics=("arbitrary",)),
        interpret=interpret,
    )(x, gate_w)

    return (
        probs_t.T.reshape(b, s, NUM_EXPERTS),
        idx_t.T.reshape(b, s, TOP_K),
        wts_t.T.reshape(b, s, TOP_K),
    )


# speedup vs baseline: 1.2525x; 1.0262x over previous
import functools

import jax
import jax.numpy as jnp
from jax.experimental import pallas as pl
from jax.experimental.pallas import tpu as pltpu

HIDDEN = 2048
NUM_EXPERTS = 8
TOP_K = 2
BLOCK_ROWS = 1024


def _router_block(x_ref, w_ref, probs_ref, idx_ref, wts_ref):
    logits_t = jax.lax.dot_general(
        w_ref[...], x_ref[...],
        dimension_numbers=(((1,), (1,)), ((), ())),
        preferred_element_type=jnp.float32,
    )
    m = jnp.max(logits_t, axis=0, keepdims=True)
    e = jnp.exp(logits_t - m)
    s = jnp.sum(e, axis=0, keepdims=True)
    probs_t = e / s

    iota = jax.lax.broadcasted_iota(jnp.int32, probs_t.shape, 0)
    v1 = jnp.max(probs_t, axis=0, keepdims=True)
    i1 = jnp.min(jnp.where(probs_t == v1, iota, NUM_EXPERTS), axis=0,
                 keepdims=True)
    masked = jnp.where(iota == i1, -jnp.inf, probs_t)
    v2 = jnp.max(masked, axis=0, keepdims=True)
    i2 = jnp.min(jnp.where(masked == v2, iota, NUM_EXPERTS), axis=0,
                 keepdims=True)

    probs_ref[...] = probs_t
    idx_ref[...] = jnp.concatenate([i1, i2], axis=0)
    denom = v1 + v2
    wts_ref[...] = jnp.concatenate([v1 / denom, v2 / denom], axis=0)


@functools.partial(jax.jit, static_argnames=("interpret",))
def kernel(hidden_states, gate_w, interpret=False):
    b, s, h = hidden_states.shape
    n = b * s
    x = hidden_states.reshape(n, h)

    grid = (n // BLOCK_ROWS,)
    probs_t, idx_t, wts_t = pl.pallas_call(
        _router_block,
        grid=grid,
        in_specs=[
            pl.BlockSpec((BLOCK_ROWS, h), lambda i: (i, 0)),
            pl.BlockSpec((NUM_EXPERTS, h), lambda i: (0, 0)),
        ],
        out_specs=[
            pl.BlockSpec((NUM_EXPERTS, BLOCK_ROWS), lambda i: (0, i)),
            pl.BlockSpec((TOP_K, BLOCK_ROWS), lambda i: (0, i)),
            pl.BlockSpec((TOP_K, BLOCK_ROWS), lambda i: (0, i)),
        ],
        out_shape=[
            jax.ShapeDtypeStruct((NUM_EXPERTS, n), jnp.float32),
            jax.ShapeDtypeStruct((TOP_K, n), jnp.int32),
            jax.ShapeDtypeStruct((TOP_K, n), jnp.float32),
        ],
        compiler_params=pltpu.CompilerParams(
            dimension_semantics=("parallel",)),
        interpret=interpret,
    )(x, gate_w)

    return (
        probs_t.T.reshape(b, s, NUM_EXPERTS),
        idx_t.T.reshape(b, s, TOP_K),
        wts_t.T.reshape(b, s, TOP_K),
    )
